# baseline (speedup 1.0000x reference)
import jax
import jax.numpy as jnp
from jax import lax
from jax.experimental import pallas as pl
from jax.experimental.pallas import tpu as pltpu
from jax.experimental.pallas import tpu_sc as plsc

N_SEQ = 4
SEQ_LEN = 1000
F_DIM = 5143 * 3
I_DIM = 300 * 3
O_DIM = F_DIM + I_DIM
L = 16


def _sc_body(f_hbm, i_hbm, its_hbm, sf_hbm, out_hbm,
             its_s, sf_s, rowf_v, rowi_v, buf_v, semf, semi):
    c = lax.axis_index("c")
    s = lax.axis_index("s")
    wid = s * 2 + c

    @pl.when(wid == 0)
    def _():
        pltpu.sync_copy(its_hbm, its_s)
        pltpu.sync_copy(sf_hbm, sf_s)
        sfv = sf_s[...]
        itsv = its_s[...]
        seq = sfv[0]
        frame = sfv[1]
        idx = itsv[N_SEQ - 1]
        for k in range(N_SEQ - 2, -1, -1):
            idx = jnp.where(seq == k, itsv[k], idx)

        cf = pltpu.async_copy(f_hbm.at[idx, pl.ds(frame, 1), :], rowf_v, semf)
        ci = pltpu.async_copy(i_hbm.at[idx, pl.ds(frame, 1), :], rowi_v, semi)
        cf.wait()
        ci.wait()

        nf = F_DIM // L
        def copy_f(i, carry):
            buf_v[0, pl.ds(i * L, L)] = rowf_v[0, pl.ds(i * L, L)]
            return carry
        lax.fori_loop(0, nf, copy_f, 0)
        buf_v[0, pl.ds(F_DIM - L, L)] = rowf_v[0, pl.ds(F_DIM - L, L)]

        ni = I_DIM // L
        def copy_i(i, carry):
            buf_v[0, pl.ds(F_DIM + i * L, L)] = rowi_v[0, pl.ds(i * L, L)]
            return carry
        lax.fori_loop(0, ni, copy_i, 0)
        buf_v[0, pl.ds(O_DIM - L, L)] = rowi_v[0, pl.ds(I_DIM - L, L)]

        pltpu.sync_copy(buf_v, out_hbm)


def kernel(flame_books, inner_books, idx_to_sequence, sequence, frame):
    its16 = jnp.pad(idx_to_sequence.astype(jnp.int32), (0, 16 - N_SEQ))
    sf16 = jnp.full((16,), jnp.asarray(frame, jnp.int32)).at[0].set(
        jnp.asarray(sequence, jnp.int32))

    mesh = plsc.VectorSubcoreMesh(core_axis_name="c", subcore_axis_name="s")
    out = pl.kernel(
        _sc_body,
        out_type=jax.ShapeDtypeStruct((1, O_DIM), jnp.float32),
        mesh=mesh,
        scratch_types=[
            pltpu.VMEM((16,), jnp.int32),
            pltpu.VMEM((16,), jnp.int32),
            pltpu.VMEM((1, F_DIM), jnp.float32),
            pltpu.VMEM((1, I_DIM), jnp.float32),
            pltpu.VMEM((1, O_DIM), jnp.float32),
            pltpu.SemaphoreType.DMA,
            pltpu.SemaphoreType.DMA,
        ],
    )(flame_books, inner_books, its16, sf16)

    return out.reshape(-1, 3)


# speedup vs baseline: 1.0570x; 1.0570x over previous
import jax
import jax.numpy as jnp
from jax import lax
from jax.experimental import pallas as pl
from jax.experimental.pallas import tpu as pltpu
from jax.experimental.pallas import tpu_sc as plsc

N_SEQ = 4
SEQ_LEN = 1000
F_DIM = 5143 * 3
I_DIM = 300 * 3


def _sc_body(f2_hbm, i2_hbm, its_hbm, sf_hbm, outf_hbm, outi_hbm,
             its_s, sf_s, rowf_v, rowi_v):
    c = lax.axis_index("c")
    s = lax.axis_index("s")
    wid = s * 2 + c

    @pl.when(wid < 2)
    def _():
        pltpu.sync_copy(its_hbm, its_s)
        pltpu.sync_copy(sf_hbm, sf_s)
        sfv = sf_s[...]
        itsv = its_s[...]
        seq = sfv[0]
        frame = sfv[1]
        idx = itsv[N_SEQ - 1]
        for k in range(N_SEQ - 2, -1, -1):
            idx = jnp.where(seq == k, itsv[k], idx)

        @pl.when(wid == 0)
        def _():
            pltpu.sync_copy(f2_hbm.at[idx, pl.ds(frame, 1), :], rowf_v)
            pltpu.sync_copy(rowf_v, outf_hbm)

        @pl.when(wid == 1)
        def _():
            pltpu.sync_copy(i2_hbm.at[idx, pl.ds(frame, 1), :], rowi_v)
            pltpu.sync_copy(rowi_v, outi_hbm)


def kernel(flame_books, inner_books, idx_to_sequence, sequence, frame):
    its16 = jnp.pad(idx_to_sequence.astype(jnp.int32), (0, 16 - N_SEQ))
    sf16 = jnp.full((16,), jnp.asarray(frame, jnp.int32)).at[0].set(
        jnp.asarray(sequence, jnp.int32))

    mesh = plsc.VectorSubcoreMesh(core_axis_name="c", subcore_axis_name="s")
    outf, outi = pl.kernel(
        _sc_body,
        out_type=[
            jax.ShapeDtypeStruct((1, F_DIM), jnp.float32),
            jax.ShapeDtypeStruct((1, I_DIM), jnp.float32),
        ],
        mesh=mesh,
        scratch_types=[
            pltpu.VMEM((16,), jnp.int32),
            pltpu.VMEM((16,), jnp.int32),
            pltpu.VMEM((1, F_DIM), jnp.float32),
            pltpu.VMEM((1, I_DIM), jnp.float32),
        ],
    )(flame_books, inner_books, its16, sf16)
    return jnp.concatenate(
        [outf.reshape(-1, 3), outi.reshape(-1, 3)], axis=0
    )
